# baseline (speedup 1.0000x reference)
import functools

import jax
import jax.numpy as jnp
from jax import lax
from jax.experimental import pallas as pl
from jax.experimental.pallas import tpu as pltpu
from jax.experimental.pallas import tpu_sc as plsc

B, C, H, W = 4, 19, 512, 512
NPIX = B * H * W
K_BINS = 128
SLOTS = 2 * K_BINS
CSLOTS = C * SLOTS
HIST = CSLOTS * 16
NC, NS = 2, 16
NW = NC * NS
TOTAL = C * NPIX
PER_TILE = TOTAL // NW
CHUNK = 16384
NCHUNKS = PER_TILE // CHUNK
HB = 64


def _phase1_body(logits_ref, tgt_ref, out_ref):
    l = logits_ref[...]
    t = tgt_ref[...]
    m = jnp.max(l, axis=1, keepdims=True)
    e = jnp.exp(l - m)
    recip = K_BINS / jnp.sum(e, axis=1, keepdims=True)
    tq = e * recip
    cls = lax.broadcasted_iota(jnp.int32, (1, C, HB, W), 1)
    pos = t[:, None, :, :] == cls
    u = jnp.where(pos, 2.0 * K_BINS - tq, tq)
    pos_i = pos.astype(jnp.int32)
    slot = jnp.minimum(u.astype(jnp.int32), pos_i * K_BINS + (K_BINS - 1))
    out_ref[...] = cls * SLOTS + slot


def _phase1(logits, targets):
    return pl.pallas_call(
        _phase1_body,
        grid=(B, H // HB),
        in_specs=[
            pl.BlockSpec((1, C, HB, W), lambda b, h: (b, 0, h, 0)),
            pl.BlockSpec((1, HB, W), lambda b, h: (b, h, 0)),
        ],
        out_specs=pl.BlockSpec((1, C, HB, W), lambda b, h: (b, 0, h, 0)),
        out_shape=jax.ShapeDtypeStruct((B, C, H, W), jnp.int32),
    )(logits, targets)


UNROLL = 8


def _phase2_body(addr_hbm, out_hbm, hist, red, stage0, stage1, sem0, sem1):
    wid = lax.axis_index("s") * NC + lax.axis_index("c")
    base = wid * PER_TILE
    stages = (stage0, stage1)
    sems = (sem0, sem1)

    zeros = jnp.zeros((16,), jnp.float32)

    def zbody(i, carry):
        hist[pl.ds(i * 16, 16)] = zeros
        return carry

    lax.fori_loop(0, HIST // 16, zbody, 0)

    ones = jnp.ones((16,), jnp.float32)
    lane = lax.iota(jnp.int32, 16)

    def start_dma(k, b):
        return pltpu.async_copy(
            addr_hbm.at[pl.ds(base + k * CHUNK, CHUNK)], stages[b], sems[b])

    start_dma(0, 0)

    def outer(g, carry):
        for b in range(2):
            k = g * 2 + b
            pltpu.make_async_copy(
                addr_hbm.at[pl.ds(base, CHUNK)], stages[b], sems[b]).wait()

            @pl.when(k + 1 < NCHUNKS)
            def _():
                start_dma(k + 1, 1 - b)

            stage = stages[b]

            def ibody(i, c2):
                for u in range(UNROLL):
                    a = stage[pl.ds((i * UNROLL + u) * 16, 16)]
                    plsc.addupdate_scatter(
                        hist, [(a << 4) + lane], ones)
                return c2

            lax.fori_loop(0, CHUNK // (16 * UNROLL), ibody, 0)
        return carry

    lax.fori_loop(0, NCHUNKS // 2, outer, 0)

    last = lane == 15

    def rbody(i, carry):
        for u in range(8):
            s = i * 8 + u
            v = plsc.cumsum(hist[pl.ds(s * 16, 16)])
            plsc.store_scatter(
                red, [jnp.full((16,), s, jnp.int32)], v, mask=last)
        return carry

    lax.fori_loop(0, CSLOTS // 8, rbody, 0)

    pltpu.sync_copy(red, out_hbm.at[wid])


@functools.cache
def _phase2():
    return pl.kernel(
        _phase2_body,
        mesh=plsc.VectorSubcoreMesh(
            core_axis_name="c", subcore_axis_name="s", num_cores=NC,
            num_subcores=NS),
        out_type=jax.ShapeDtypeStruct((NW, CSLOTS), jnp.float32),
        compiler_params=pltpu.CompilerParams(needs_layout_passes=False),
        scratch_types=[
            pltpu.VMEM((HIST,), jnp.float32),
            pltpu.VMEM((CSLOTS,), jnp.float32),
            pltpu.VMEM((CHUNK,), jnp.int32),
            pltpu.VMEM((CHUNK,), jnp.int32),
            pltpu.SemaphoreType.DMA,
            pltpu.SemaphoreType.DMA,
        ],
    )


def _cumsum_lanes(x):
    n = x.shape[-1]
    k = 1
    while k < n:
        shifted = jnp.concatenate(
            [jnp.zeros(x.shape[:-1] + (k,), x.dtype), x[..., :-k]], axis=-1)
        x = x + shifted
        k *= 2
    return x


def _phase3_body(part_ref, out_ref):
    hs = jnp.sum(part_ref[...], axis=0)
    m = hs[:, :K_BINS] + hs[:, K_BINS:]
    q = hs[:, K_BINS:]
    cm = _cumsum_lanes(m)
    sq = _cumsum_lanes(q)
    ntot = cm[:, K_BINS - 1:K_BINS]
    p_tot = sq[:, K_BINS - 1:K_BINS]
    n_gt = ntot - cm
    n_ge = n_gt + m
    s_gt = p_tot - sq
    s_ge = s_gt + q
    j_ge = jnp.where(n_ge > 0, n_ge / (p_tot + n_ge - s_ge), 0.0)
    j_gt = jnp.where(n_gt > 0, n_gt / (p_tot + n_gt - s_gt), 0.0)
    v = (lax.broadcasted_iota(jnp.int32, (C, K_BINS), 1).astype(jnp.float32)
         + 0.5) / K_BINS
    loss_c = jnp.sum(v * (j_ge - j_gt), axis=1, keepdims=True)
    present = (p_tot > 0).astype(jnp.float32)
    total = jnp.sum(jnp.where(p_tot > 0, loss_c, 0.0), axis=0, keepdims=True)
    n_pres = jnp.sum(present, axis=0, keepdims=True)
    out_ref[...] = jnp.where(
        n_pres > 0, total / jnp.maximum(n_pres, 1.0), 0.0)


def _phase3(partials):
    return pl.pallas_call(
        _phase3_body,
        out_shape=jax.ShapeDtypeStruct((1, 1), jnp.float32),
    )(partials)


def kernel(logits, targets):
    addr = _phase1(logits, targets.astype(jnp.int32))
    partials = _phase2()(addr.reshape(-1))
    return _phase3(partials.reshape(NW, C, SLOTS))[0, 0]


# speedup vs baseline: 1.6155x; 1.6155x over previous
import functools

import jax
import jax.numpy as jnp
from jax import lax
from jax.experimental import pallas as pl
from jax.experimental.pallas import tpu as pltpu
from jax.experimental.pallas import tpu_sc as plsc

B, C, H, W = 4, 19, 512, 512
NPIX = B * H * W
K_BINS = 512
SLOTS = 2 * K_BINS
HIST = C * SLOTS
BANKS = 4
NC, NS = 2, 16
NW = NC * NS
TOTAL = C * NPIX
TOTALW = TOTAL // 2
PER_TILE = TOTALW // NW
CHUNK = 8192
NCHUNKS = PER_TILE // CHUNK
HB = 64


def _phase1_body(logits_ref, tgt_ref, out_ref):
    l = logits_ref[...]
    t = tgt_ref[...]
    m = jnp.max(l, axis=1, keepdims=True)
    e = jnp.exp(l - m)
    recip = K_BINS / jnp.sum(e, axis=1, keepdims=True)
    tq = e * recip
    cls = lax.broadcasted_iota(jnp.int32, (1, C, HB, W), 1)
    pos = t[:, None, :, :] == cls
    u = jnp.where(pos, 2.0 * K_BINS - tq, tq)
    pos_i = pos.astype(jnp.int32)
    slot = jnp.minimum(u.astype(jnp.int32), pos_i * K_BINS + (K_BINS - 1))
    a = cls * SLOTS + slot
    out_ref[...] = a[:, :, :HB // 2, :] | (a[:, :, HB // 2:, :] << 16)


def _phase1(logits, targets):
    return pl.pallas_call(
        _phase1_body,
        grid=(B, H // HB),
        in_specs=[
            pl.BlockSpec((1, C, HB, W), lambda b, h: (b, 0, h, 0)),
            pl.BlockSpec((1, HB, W), lambda b, h: (b, h, 0)),
        ],
        out_specs=pl.BlockSpec((1, C, HB // 2, W), lambda b, h: (b, 0, h, 0)),
        out_shape=jax.ShapeDtypeStruct((B, C, H // 2, W), jnp.int32),
    )(logits, targets)


UNROLL = 8


def _phase2_body(addr_hbm, out_hbm, h0, h1, h2, h3, stage0, stage1,
                 sem0, sem1):
    wid = lax.axis_index("s") * NC + lax.axis_index("c")
    base = wid * PER_TILE
    banks = (h0, h1, h2, h3)
    stages = (stage0, stage1)
    sems = (sem0, sem1)

    zeros = jnp.zeros((16,), jnp.float32)

    def zbody(i, carry):
        for hb in banks:
            hb[pl.ds(i * 16, 16)] = zeros
        return carry

    lax.fori_loop(0, HIST // 16, zbody, 0)

    ones = jnp.ones((16,), jnp.float32)

    def start_dma(k, b):
        return pltpu.async_copy(
            addr_hbm.at[pl.ds(base + k * CHUNK, CHUNK)], stages[b], sems[b])

    start_dma(0, 0)

    def outer(g, carry):
        for b in range(2):
            k = g * 2 + b
            pltpu.make_async_copy(
                addr_hbm.at[pl.ds(base, CHUNK)], stages[b], sems[b]).wait()

            @pl.when(k + 1 < NCHUNKS)
            def _():
                start_dma(k + 1, 1 - b)

            stage = stages[b]

            def ibody(i, c2):
                for u in range(UNROLL):
                    w = stage[pl.ds((i * UNROLL + u) * 16, 16)]
                    lo = w & 0xFFFF
                    hi = w >> 16
                    plsc.addupdate_scatter(banks[(2 * u) % BANKS], [lo], ones)
                    plsc.addupdate_scatter(
                        banks[(2 * u + 1) % BANKS], [hi], ones)
                return c2

            lax.fori_loop(0, CHUNK // (16 * UNROLL), ibody, 0)
        return carry

    lax.fori_loop(0, NCHUNKS // 2, outer, 0)

    def mbody(i, carry):
        sl = pl.ds(i * 16, 16)
        h0[sl] = (h0[sl] + h1[sl]) + (h2[sl] + h3[sl])
        return carry

    lax.fori_loop(0, HIST // 16, mbody, 0)

    pltpu.sync_copy(h0, out_hbm.at[wid])


@functools.cache
def _phase2():
    return pl.kernel(
        _phase2_body,
        mesh=plsc.VectorSubcoreMesh(
            core_axis_name="c", subcore_axis_name="s", num_cores=NC,
            num_subcores=NS),
        out_type=jax.ShapeDtypeStruct((NW, HIST), jnp.float32),
        compiler_params=pltpu.CompilerParams(needs_layout_passes=False),
        scratch_types=[
            pltpu.VMEM((HIST,), jnp.float32),
            pltpu.VMEM((HIST,), jnp.float32),
            pltpu.VMEM((HIST,), jnp.float32),
            pltpu.VMEM((HIST,), jnp.float32),
            pltpu.VMEM((CHUNK,), jnp.int32),
            pltpu.VMEM((CHUNK,), jnp.int32),
            pltpu.SemaphoreType.DMA,
            pltpu.SemaphoreType.DMA,
        ],
    )


def _cumsum_lanes(x):
    n = x.shape[-1]
    k = 1
    while k < n:
        shifted = jnp.concatenate(
            [jnp.zeros(x.shape[:-1] + (k,), x.dtype), x[..., :-k]], axis=-1)
        x = x + shifted
        k *= 2
    return x


def _phase3_body(part_ref, out_ref):
    hs = jnp.sum(part_ref[...], axis=0)
    m = hs[:, :K_BINS] + hs[:, K_BINS:]
    q = hs[:, K_BINS:]
    cm = _cumsum_lanes(m)
    sq = _cumsum_lanes(q)
    ntot = cm[:, K_BINS - 1:K_BINS]
    p_tot = sq[:, K_BINS - 1:K_BINS]
    n_gt = ntot - cm
    n_ge = n_gt + m
    s_gt = p_tot - sq
    s_ge = s_gt + q
    j_ge = jnp.where(n_ge > 0, n_ge / (p_tot + n_ge - s_ge), 0.0)
    j_gt = jnp.where(n_gt > 0, n_gt / (p_tot + n_gt - s_gt), 0.0)
    v = (lax.broadcasted_iota(jnp.int32, (C, K_BINS), 1).astype(jnp.float32)
         + 0.5) / K_BINS
    loss_c = jnp.sum(v * (j_ge - j_gt), axis=1, keepdims=True)
    present = (p_tot > 0).astype(jnp.float32)
    total = jnp.sum(jnp.where(p_tot > 0, loss_c, 0.0), axis=0, keepdims=True)
    n_pres = jnp.sum(present, axis=0, keepdims=True)
    out_ref[...] = jnp.where(
        n_pres > 0, total / jnp.maximum(n_pres, 1.0), 0.0)


def _phase3(partials):
    return pl.pallas_call(
        _phase3_body,
        out_shape=jax.ShapeDtypeStruct((1, 1), jnp.float32),
    )(partials)


def kernel(logits, targets):
    addr = _phase1(logits, targets.astype(jnp.int32))
    partials = _phase2()(addr.reshape(-1))
    return _phase3(partials.reshape(NW, C, SLOTS))[0, 0]
